# baseline (speedup 1.0000x reference)
import functools

import jax
import jax.numpy as jnp
from jax import lax
from jax.experimental import pallas as pl
from jax.experimental.pallas import tpu as pltpu

S, D = 2048, 1024
NH, NKV, HD = 16, 4, 64
NREP = NH // NKV
E, TOPK = 8, 2
HID = 4 * D // 3
HIDP = 1408
CAP = int(S * TOPK / E * 1.25)
XSTRIDE = CAP + 8
NROWS = E * XSTRIDE
EPS = 1e-06
SB = 512
QB = 1024


def _k1_body(x_ref, wq_ref, wk_ref, wv_ref, nw_ref, cq_ref, sq_ref, ck_ref,
             sk_ref, rq_ref, rk_ref, q_ref, k_ref, v_ref):
    x = x_ref[...]
    ms = jnp.mean(x * x, axis=-1, keepdims=True)
    h = x * lax.rsqrt(ms + EPS) * nw_ref[...]
    q = jnp.dot(h, wq_ref[...], preferred_element_type=jnp.float32)
    k = jnp.dot(h, wk_ref[...], preferred_element_type=jnp.float32)
    v = jnp.dot(h, wv_ref[...], preferred_element_type=jnp.float32)
    rotq = jnp.dot(q.astype(jnp.bfloat16), rq_ref[...],
                   preferred_element_type=jnp.float32)
    rotk = jnp.dot(k.astype(jnp.bfloat16), rk_ref[...],
                   preferred_element_type=jnp.float32)
    q_ref[...] = q * cq_ref[...] + rotq * sq_ref[...]
    k_ref[...] = k * ck_ref[...] + rotk * sk_ref[...]
    v_ref[...] = v


def _k1(x, wq_p, wk_p, wv, nw, cq, sq, ck, sk, rq, rk):
    grid = (S // SB,)
    return pl.pallas_call(
        _k1_body,
        grid=grid,
        in_specs=[
            pl.BlockSpec((SB, D), lambda i: (i, 0)),
            pl.BlockSpec((D, NH * HD), lambda i: (0, 0)),
            pl.BlockSpec((D, NKV * HD), lambda i: (0, 0)),
            pl.BlockSpec((D, NKV * HD), lambda i: (0, 0)),
            pl.BlockSpec((1, D), lambda i: (0, 0)),
            pl.BlockSpec((SB, NH * HD), lambda i: (i, 0)),
            pl.BlockSpec((SB, NH * HD), lambda i: (i, 0)),
            pl.BlockSpec((SB, NKV * HD), lambda i: (i, 0)),
            pl.BlockSpec((SB, NKV * HD), lambda i: (i, 0)),
            pl.BlockSpec((NH * HD, NH * HD), lambda i: (0, 0)),
            pl.BlockSpec((NKV * HD, NKV * HD), lambda i: (0, 0)),
        ],
        out_specs=[
            pl.BlockSpec((SB, NH * HD), lambda i: (i, 0)),
            pl.BlockSpec((SB, NKV * HD), lambda i: (i, 0)),
            pl.BlockSpec((SB, NKV * HD), lambda i: (i, 0)),
        ],
        out_shape=[
            jax.ShapeDtypeStruct((S, NH * HD), jnp.float32),
            jax.ShapeDtypeStruct((S, NKV * HD), jnp.float32),
            jax.ShapeDtypeStruct((S, NKV * HD), jnp.float32),
        ],
    )(x, wq_p, wk_p, wv, nw, cq, sq, ck, sk, rq, rk)


def _k2_body(q_ref, k_ref, v_ref, o_ref):
    p = pl.program_id(0)
    use_low = (p % 4) < 2
    kb = k_ref[...]
    vb = v_ref[...]
    khb = jnp.where(use_low, kb[:, :HD], kb[:, HD:]).astype(jnp.bfloat16)
    vhb = jnp.where(use_low, vb[:, :HD], vb[:, HD:]).astype(jnp.bfloat16)
    q = q_ref[...]
    outs = []
    for t in range(2):
        qt = q[:, t * HD:(t + 1) * HD].astype(jnp.bfloat16)
        s = lax.dot_general(qt, khb, (((1,), (1,)), ((), ())),
                            preferred_element_type=jnp.float32) * (1.0 / 8.0)
        pexp = jnp.exp(s.astype(jnp.bfloat16))
        l = jnp.sum(pexp.astype(jnp.float32), axis=-1, keepdims=True)
        o = jnp.dot(pexp, vhb, preferred_element_type=jnp.float32)
        outs.append(o / l)
    o_ref[...] = jnp.concatenate(outs, axis=1)


def _k2(q_hm, k_hm, v_hm):
    grid = (NH // 2, S // QB)
    return pl.pallas_call(
        _k2_body,
        grid=grid,
        in_specs=[
            pl.BlockSpec((QB, 2 * HD), lambda p, i: (i, p)),
            pl.BlockSpec((S, 2 * HD), lambda p, i: (0, p // 4)),
            pl.BlockSpec((S, 2 * HD), lambda p, i: (0, p // 4)),
        ],
        out_specs=pl.BlockSpec((QB, 2 * HD), lambda p, i: (i, p)),
        out_shape=jax.ShapeDtypeStruct((S, NH * HD), jnp.float32),
    )(q_hm, k_hm, v_hm)


def _k3_body(ao_ref, x_ref, wo_ref, nw_ref, rw_ref, h_ref, xn_ref, lg_ref):
    ao = ao_ref[...]
    h_out = x_ref[...] + jnp.dot(ao, wo_ref[...],
                                 preferred_element_type=jnp.float32)
    h_ref[...] = h_out
    ms = jnp.mean(h_out * h_out, axis=-1, keepdims=True)
    xn = h_out * lax.rsqrt(ms + EPS) * nw_ref[...]
    xn_ref[...] = xn
    lg_ref[...] = jnp.dot(xn, rw_ref[...], preferred_element_type=jnp.float32)


def _k3(ao, x, wo, nw, rw_pad):
    grid = (S // SB,)
    return pl.pallas_call(
        _k3_body,
        grid=grid,
        in_specs=[
            pl.BlockSpec((SB, D), lambda i: (i, 0)),
            pl.BlockSpec((SB, D), lambda i: (i, 0)),
            pl.BlockSpec((D, D), lambda i: (0, 0)),
            pl.BlockSpec((1, D), lambda i: (0, 0)),
            pl.BlockSpec((D, 128), lambda i: (0, 0)),
        ],
        out_specs=[
            pl.BlockSpec((SB, D), lambda i: (i, 0)),
            pl.BlockSpec((SB, D), lambda i: (i, 0)),
            pl.BlockSpec((SB, 128), lambda i: (i, 0)),
        ],
        out_shape=[
            jax.ShapeDtypeStruct((S, D), jnp.float32),
            jax.ShapeDtypeStruct((S, D), jnp.float32),
            jax.ShapeDtypeStruct((S, 128), jnp.float32),
        ],
    )(ao, x, wo, nw, rw_pad)


def _k4_body(lg_ref, w0_ref, w1_ref, rt_ref, st_ref):
    lg = lg_ref[...]
    lane = lax.broadcasted_iota(jnp.int32, (S, 128), 1)
    valid = lane < E
    lgm = jnp.where(valid, lg, -1e30)
    m = jnp.max(lgm, axis=-1, keepdims=True)
    p = jnp.where(valid, jnp.exp(lgm - m), 0.0)
    probs = p / jnp.sum(p, axis=-1, keepdims=True)
    a0 = jnp.max(probs, axis=-1, keepdims=True)
    i0 = jnp.min(jnp.where(probs == a0, lane, 9999), axis=-1, keepdims=True)
    oh0 = lane == i0
    pm = jnp.where(oh0, -1.0, probs)
    a1 = jnp.max(pm, axis=-1, keepdims=True)
    i1 = jnp.min(jnp.where(pm == a1, lane, 9999), axis=-1, keepdims=True)
    oh1 = lane == i1
    wsum = a0 + a1
    w0 = a0 / wsum
    w1 = a1 / wsum
    ind = jnp.logical_or(oh0, oh1)
    indf = ind.astype(jnp.float32)
    r_io = lax.broadcasted_iota(jnp.int32, (S, S), 0)
    c_io = lax.broadcasted_iota(jnp.int32, (S, S), 1)
    ltri = (r_io > c_io).astype(jnp.bfloat16)
    rank = jnp.dot(ltri, indf.astype(jnp.bfloat16),
                   preferred_element_type=jnp.float32)
    oh0f = oh0.astype(jnp.float32)
    oh1f = oh1.astype(jnp.float32)
    r0 = jnp.sum(rank * oh0f, axis=-1, keepdims=True)
    r1 = jnp.sum(rank * oh1f, axis=-1, keepdims=True)
    v0 = r0 < CAP
    v1 = r1 < CAP
    base0 = i0 * XSTRIDE
    base1 = i1 * XSTRIDE
    f0 = jnp.where(v0, base0 + r0.astype(jnp.int32), base0 + CAP)
    f1 = jnp.where(v1, base1 + r1.astype(jnp.int32), base1 + CAP)
    sw0 = jnp.where(v0, w0, 0.0)
    sw1 = jnp.where(v1, w1, 0.0)
    w0_ref[...] = jnp.broadcast_to(sw0, (S, 128))
    w1_ref[...] = jnp.broadcast_to(sw1, (S, 128))
    pk = jnp.where(lane == 0, f0.astype(jnp.float32),
                   jnp.where(lane == 1, f1.astype(jnp.float32), 0.0))
    rt_ref[...] = jnp.transpose(pk)[:8, :]
    counts = jnp.sum(indf, axis=0, keepdims=True)
    psum = jnp.sum(probs, axis=0, keepdims=True)
    nt = float(S)
    aux = E * jnp.sum(psum / nt * (counts / nt), axis=-1, keepdims=True)
    util = 100.0 * jnp.sum((counts > 0).astype(jnp.float32), axis=-1,
                           keepdims=True) / E
    row = lax.broadcasted_iota(jnp.int32, (8, 128), 0)
    st_ref[...] = jnp.where(row == 0, jnp.broadcast_to(aux, (8, 128)),
                            jnp.broadcast_to(util, (8, 128)))


def _k4(logits):
    return pl.pallas_call(
        _k4_body,
        out_shape=[
            jax.ShapeDtypeStruct((S, 128), jnp.float32),
            jax.ShapeDtypeStruct((S, 128), jnp.float32),
            jax.ShapeDtypeStruct((8, S), jnp.float32),
            jax.ShapeDtypeStruct((8, 128), jnp.float32),
        ],
    )(logits)


def _k6_body(x_ref, w1_ref, w2_ref, w3_ref, ws_ref, o_ref):
    x = x_ref[...]
    a = jnp.dot(x, w1_ref[0], preferred_element_type=jnp.float32)
    b = jnp.dot(x, w3_ref[0], preferred_element_type=jnp.float32)
    hsw = a * jax.nn.sigmoid(a) * b
    y = jnp.dot(hsw, w2_ref[0], preferred_element_type=jnp.float32)
    o_ref[...] = y * ws_ref[:, :1]


def _k6(xe, e_w1, e_w2, e_w3, w_slot):
    grid = (E,)
    return pl.pallas_call(
        _k6_body,
        grid=grid,
        in_specs=[
            pl.BlockSpec((XSTRIDE, D), lambda e: (e, 0)),
            pl.BlockSpec((1, D, HID), lambda e: (e, 0, 0)),
            pl.BlockSpec((1, HID, D), lambda e: (e, 0, 0)),
            pl.BlockSpec((1, D, HID), lambda e: (e, 0, 0)),
            pl.BlockSpec((XSTRIDE, 128), lambda e: (e, 0)),
        ],
        out_specs=pl.BlockSpec((XSTRIDE, D), lambda e: (e, 0)),
        out_shape=jax.ShapeDtypeStruct((NROWS, D), jnp.float32),
    )(xe, e_w1, e_w2, e_w3, w_slot)


def _ks_body(xn_ref, h_ref, w1_ref, w2_ref, w3_ref, o_ref):
    xn = xn_ref[...]
    a = jnp.dot(xn, w1_ref[...], preferred_element_type=jnp.float32)
    b = jnp.dot(xn, w3_ref[...], preferred_element_type=jnp.float32)
    hsw = a * jax.nn.sigmoid(a) * b
    y = jnp.dot(hsw, w2_ref[...], preferred_element_type=jnp.float32)
    o_ref[...] = h_ref[...] + y


def _ks(xn, h_out, s_w1, s_w2, s_w3):
    grid = (S // SB,)
    return pl.pallas_call(
        _ks_body,
        grid=grid,
        in_specs=[
            pl.BlockSpec((SB, D), lambda i: (i, 0)),
            pl.BlockSpec((SB, D), lambda i: (i, 0)),
            pl.BlockSpec((D, HID), lambda i: (0, 0)),
            pl.BlockSpec((HID, D), lambda i: (0, 0)),
            pl.BlockSpec((D, HID), lambda i: (0, 0)),
        ],
        out_specs=pl.BlockSpec((SB, D), lambda i: (i, 0)),
        out_shape=jax.ShapeDtypeStruct((S, D), jnp.float32),
    )(xn, h_out, s_w1, s_w2, s_w3)


def _k9_body(f_ref, m0_ref, m1_ref, o_ref):
    o_ref[...] = f_ref[...] + m0_ref[...] + m1_ref[...]


def _k9(fin, m0, m1):
    grid = (S // SB,)
    return pl.pallas_call(
        _k9_body,
        grid=grid,
        in_specs=[
            pl.BlockSpec((SB, D), lambda i: (i, 0)),
            pl.BlockSpec((SB, D), lambda i: (i, 0)),
            pl.BlockSpec((SB, D), lambda i: (i, 0)),
        ],
        out_specs=pl.BlockSpec((SB, D), lambda i: (i, 0)),
        out_shape=jax.ShapeDtypeStruct((S, D), jnp.float32),
    )(fin, m0, m1)


from jax.experimental.pallas import tpu_sc as plsc

NSC, NSUB = 2, 16
NW = NSC * NSUB
TPW = S // NW
NG = TPW // 16


def _sc_mesh():
    return plsc.VectorSubcoreMesh(core_axis_name="c", subcore_axis_name="s",
                                  num_cores=NSC, num_subcores=NSUB)


def _load_idx(rt_hbm, row, base, r_v, i_v):
    pltpu.sync_copy(rt_hbm.at[pl.ds(row * S + base, TPW)], r_v)
    for g in range(NG):
        i_v[pl.ds(g * 16, 16)] = r_v[pl.ds(g * 16, 16)].astype(jnp.int32)


def _sc_dispatch_body(xn_hbm, w0_hbm, w1_hbm, rt_hbm, xe_hbm, ws_hbm,
                      xn_v, w0_v, w1_v, r_v, i0_v, i1_v, sem):
    wid = lax.axis_index("s") * NSC + lax.axis_index("c")
    base = wid * TPW
    pltpu.sync_copy(xn_hbm.at[pl.ds(base, TPW)], xn_v)
    pltpu.sync_copy(w0_hbm.at[pl.ds(base, TPW)], w0_v)
    pltpu.sync_copy(w1_hbm.at[pl.ds(base, TPW)], w1_v)
    _load_idx(rt_hbm, 0, base, r_v, i0_v)
    _load_idx(rt_hbm, 1, base, r_v, i1_v)
    pltpu.async_copy(xn_v, xe_hbm.at[i0_v], sem).wait()
    pltpu.async_copy(xn_v, xe_hbm.at[i1_v], sem).wait()
    pltpu.async_copy(w0_v, ws_hbm.at[i0_v], sem).wait()
    pltpu.async_copy(w1_v, ws_hbm.at[i1_v], sem).wait()


def _dispatch(xn, w0b, w1b, rt_flat):
    return pl.kernel(
        _sc_dispatch_body,
        out_type=[
            jax.ShapeDtypeStruct((NROWS, D), jnp.float32),
            jax.ShapeDtypeStruct((NROWS, 128), jnp.float32),
        ],
        mesh=_sc_mesh(),
        scratch_types=[
            pltpu.VMEM((TPW, D), jnp.float32),
            pltpu.VMEM((TPW, 128), jnp.float32),
            pltpu.VMEM((TPW, 128), jnp.float32),
            pltpu.VMEM((TPW,), jnp.float32),
            pltpu.VMEM((TPW,), jnp.int32),
            pltpu.VMEM((TPW,), jnp.int32),
            pltpu.SemaphoreType.DMA,
        ],
    )(xn, w0b, w1b, rt_flat)


def _sc_combine_body(ye_hbm, rt_hbm, m0_hbm, m1_hbm,
                     r_v, i0_v, i1_v, g_v, sem):
    wid = lax.axis_index("s") * NSC + lax.axis_index("c")
    base = wid * TPW
    _load_idx(rt_hbm, 0, base, r_v, i0_v)
    _load_idx(rt_hbm, 1, base, r_v, i1_v)
    half = TPW // 2
    for h in range(2):
        pltpu.async_copy(ye_hbm.at[i0_v.at[pl.ds(h * half, half)]], g_v,
                         sem).wait()
        pltpu.sync_copy(g_v, m0_hbm.at[pl.ds(base + h * half, half)])
        pltpu.async_copy(ye_hbm.at[i1_v.at[pl.ds(h * half, half)]], g_v,
                         sem).wait()
        pltpu.sync_copy(g_v, m1_hbm.at[pl.ds(base + h * half, half)])


def _combine(ye, rt_flat):
    half = TPW // 2
    return pl.kernel(
        _sc_combine_body,
        out_type=[
            jax.ShapeDtypeStruct((S, D), jnp.float32),
            jax.ShapeDtypeStruct((S, D), jnp.float32),
        ],
        mesh=_sc_mesh(),
        scratch_types=[
            pltpu.VMEM((TPW,), jnp.float32),
            pltpu.VMEM((TPW,), jnp.int32),
            pltpu.VMEM((TPW,), jnp.int32),
            pltpu.VMEM((half, D), jnp.float32),
            pltpu.SemaphoreType.DMA,
        ],
    )(ye, rt_flat)


def kernel(x_in, cos, sin, mask, attn_norm_w, wq, wk, wv, wo, moe_norm_w,
           router_w, e_w1, e_w2, e_w3, s_w1, s_w2, s_w3):
    x = x_in.reshape(S, D)
    c32 = cos[0, :, 0, :32]
    s32 = sin[0, :, 0, :32]
    c64 = jnp.concatenate([c32, c32], axis=1)
    s64 = jnp.concatenate([-s32, s32], axis=1)
    cq = jnp.tile(c64, (1, NH))
    sq = jnp.tile(s64, (1, NH))
    ck = jnp.tile(c64, (1, NKV))
    sk = jnp.tile(s64, (1, NKV))
    nw_a = attn_norm_w.reshape(1, D)
    nw_m = moe_norm_w.reshape(1, D)
    rw_pad = jnp.pad(router_w, ((0, 0), (0, 128 - E)))

    ar_q = jnp.arange(NH * HD)
    rq = (
        (ar_q[:, None] ^ 32) == ar_q[None, :]).astype(jnp.bfloat16)
    ar_k = jnp.arange(NKV * HD)
    rk = (
        (ar_k[:, None] ^ 32) == ar_k[None, :]).astype(jnp.bfloat16)

    q_hm, k_hm, v_hm = _k1(x, wq, wk, wv, nw_a, cq, sq, ck, sk, rq, rk)

    new_k = k_hm.reshape(1, S, NKV, HD)
    new_v = v_hm.reshape(1, S, NKV, HD)

    ao = _k2(q_hm, k_hm, v_hm)

    h_out, xn, logits = _k3(ao, x, wo, nw_m, rw_pad)

    fin = _ks(xn, h_out, s_w1, s_w2, s_w3)

    w0b, w1b, rt, stats = _k4(logits)
    rt_flat = rt.reshape(8 * S)

    xe, w_slot = _dispatch(xn, w0b, w1b, rt_flat)

    ye = _k6(xe, e_w1, e_w2, e_w3, w_slot)

    m0, m1 = _combine(ye, rt_flat)

    out = _k9(fin, m0, m1)

    return (out.reshape(1, S, D), stats[0, 0], stats[1, 0], new_k, new_v)


# speedup vs baseline: 1.0150x; 1.0150x over previous
import jax
import jax.numpy as jnp
from jax import lax
from jax.experimental import pallas as pl
from jax.experimental.pallas import tpu as pltpu

S, D = 2048, 1024
NH, NKV, HD = 16, 4, 64
NREP = NH // NKV
E, TOPK = 8, 2
HID = 4 * D // 3
HIDP = 1408
CAP = int(S * TOPK / E * 1.25)
XSTRIDE = CAP + 8
NROWS = E * XSTRIDE
EPS = 1e-06
SB = 512
QB = 1024


def _k1_body(x_ref, wq_ref, wk_ref, wv_ref, nw_ref, cq_ref, sq_ref, ck_ref,
             sk_ref, rq_ref, rk_ref, q_ref, k_ref, v_ref):
    x = x_ref[...]
    ms = jnp.mean(x * x, axis=-1, keepdims=True)
    h = x * lax.rsqrt(ms + EPS) * nw_ref[...]
    q = jnp.dot(h, wq_ref[...], preferred_element_type=jnp.float32)
    k = jnp.dot(h, wk_ref[...], preferred_element_type=jnp.float32)
    v = jnp.dot(h, wv_ref[...], preferred_element_type=jnp.float32)
    rotq = jnp.dot(q.astype(jnp.bfloat16), rq_ref[...],
                   preferred_element_type=jnp.float32)
    rotk = jnp.dot(k.astype(jnp.bfloat16), rk_ref[...],
                   preferred_element_type=jnp.float32)
    q_ref[...] = q * cq_ref[...] + rotq * sq_ref[...]
    k_ref[...] = k * ck_ref[...] + rotk * sk_ref[...]
    v_ref[...] = v


def _k1(x, wq_p, wk_p, wv, nw, cq, sq, ck, sk, rq, rk):
    grid = (S // SB,)
    return pl.pallas_call(
        _k1_body,
        grid=grid,
        in_specs=[
            pl.BlockSpec((SB, D), lambda i: (i, 0)),
            pl.BlockSpec((D, NH * HD), lambda i: (0, 0)),
            pl.BlockSpec((D, NKV * HD), lambda i: (0, 0)),
            pl.BlockSpec((D, NKV * HD), lambda i: (0, 0)),
            pl.BlockSpec((1, D), lambda i: (0, 0)),
            pl.BlockSpec((SB, NH * HD), lambda i: (i, 0)),
            pl.BlockSpec((SB, NH * HD), lambda i: (i, 0)),
            pl.BlockSpec((SB, NKV * HD), lambda i: (i, 0)),
            pl.BlockSpec((SB, NKV * HD), lambda i: (i, 0)),
            pl.BlockSpec((NH * HD, NH * HD), lambda i: (0, 0)),
            pl.BlockSpec((NKV * HD, NKV * HD), lambda i: (0, 0)),
        ],
        out_specs=[
            pl.BlockSpec((SB, NH * HD), lambda i: (i, 0)),
            pl.BlockSpec((SB, NKV * HD), lambda i: (i, 0)),
            pl.BlockSpec((SB, NKV * HD), lambda i: (i, 0)),
        ],
        out_shape=[
            jax.ShapeDtypeStruct((S, NH * HD), jnp.float32),
            jax.ShapeDtypeStruct((S, NKV * HD), jnp.float32),
            jax.ShapeDtypeStruct((S, NKV * HD), jnp.float32),
        ],
    )(x, wq_p, wk_p, wv, nw, cq, sq, ck, sk, rq, rk)


def _k2_body(q_ref, k_ref, v_ref, o_ref):
    p = pl.program_id(0)
    use_low = (p % 4) < 2
    kb = k_ref[...]
    vb = v_ref[...]
    khb = jnp.where(use_low, kb[:, :HD], kb[:, HD:]).astype(jnp.bfloat16)
    vhb = jnp.where(use_low, vb[:, :HD], vb[:, HD:]).astype(jnp.bfloat16)
    q = q_ref[...]
    outs = []
    for t in range(2):
        qt = q[:, t * HD:(t + 1) * HD].astype(jnp.bfloat16)
        s = lax.dot_general(qt, khb, (((1,), (1,)), ((), ())),
                            preferred_element_type=jnp.float32) * (1.0 / 8.0)
        pexp = jnp.exp(s.astype(jnp.bfloat16))
        l = jnp.sum(pexp.astype(jnp.float32), axis=-1, keepdims=True)
        o = jnp.dot(pexp, vhb, preferred_element_type=jnp.float32)
        outs.append(o / l)
    o_ref[...] = jnp.concatenate(outs, axis=1)


def _k2(q_hm, k_hm, v_hm):
    grid = (NH // 2, S // QB)
    return pl.pallas_call(
        _k2_body,
        grid=grid,
        in_specs=[
            pl.BlockSpec((QB, 2 * HD), lambda p, i: (i, p)),
            pl.BlockSpec((S, 2 * HD), lambda p, i: (0, p // 4)),
            pl.BlockSpec((S, 2 * HD), lambda p, i: (0, p // 4)),
        ],
        out_specs=pl.BlockSpec((QB, 2 * HD), lambda p, i: (i, p)),
        out_shape=jax.ShapeDtypeStruct((S, NH * HD), jnp.float32),
    )(q_hm, k_hm, v_hm)


def _k3_body(ao_ref, x_ref, wo_ref, nw_ref, rw_ref, h_ref, xn_ref, lg_ref):
    ao = ao_ref[...]
    h_out = x_ref[...] + jnp.dot(ao, wo_ref[...],
                                 preferred_element_type=jnp.float32)
    h_ref[...] = h_out
    ms = jnp.mean(h_out * h_out, axis=-1, keepdims=True)
    xn = h_out * lax.rsqrt(ms + EPS) * nw_ref[...]
    xn_ref[...] = xn
    lg_ref[...] = jnp.dot(xn, rw_ref[...], preferred_element_type=jnp.float32)


def _k3(ao, x, wo, nw, rw_pad):
    grid = (S // SB,)
    return pl.pallas_call(
        _k3_body,
        grid=grid,
        in_specs=[
            pl.BlockSpec((SB, D), lambda i: (i, 0)),
            pl.BlockSpec((SB, D), lambda i: (i, 0)),
            pl.BlockSpec((D, D), lambda i: (0, 0)),
            pl.BlockSpec((1, D), lambda i: (0, 0)),
            pl.BlockSpec((D, 128), lambda i: (0, 0)),
        ],
        out_specs=[
            pl.BlockSpec((SB, D), lambda i: (i, 0)),
            pl.BlockSpec((SB, D), lambda i: (i, 0)),
            pl.BlockSpec((SB, 128), lambda i: (i, 0)),
        ],
        out_shape=[
            jax.ShapeDtypeStruct((S, D), jnp.float32),
            jax.ShapeDtypeStruct((S, D), jnp.float32),
            jax.ShapeDtypeStruct((S, 128), jnp.float32),
        ],
    )(ao, x, wo, nw, rw_pad)


def _k4_body(lg_ref, w0_ref, w1_ref, rt_ref, st_ref):
    lg = lg_ref[...]
    lane = lax.broadcasted_iota(jnp.int32, (S, 128), 1)
    valid = lane < E
    lgm = jnp.where(valid, lg, -1e30)
    m = jnp.max(lgm, axis=-1, keepdims=True)
    p = jnp.where(valid, jnp.exp(lgm - m), 0.0)
    probs = p / jnp.sum(p, axis=-1, keepdims=True)
    a0 = jnp.max(probs, axis=-1, keepdims=True)
    i0 = jnp.min(jnp.where(probs == a0, lane, 9999), axis=-1, keepdims=True)
    oh0 = lane == i0
    pm = jnp.where(oh0, -1.0, probs)
    a1 = jnp.max(pm, axis=-1, keepdims=True)
    i1 = jnp.min(jnp.where(pm == a1, lane, 9999), axis=-1, keepdims=True)
    oh1 = lane == i1
    wsum = a0 + a1
    w0 = a0 / wsum
    w1 = a1 / wsum
    ind = jnp.logical_or(oh0, oh1)
    indf = ind.astype(jnp.float32)
    r_io = lax.broadcasted_iota(jnp.int32, (S, S), 0)
    c_io = lax.broadcasted_iota(jnp.int32, (S, S), 1)
    ltri = (r_io > c_io).astype(jnp.bfloat16)
    rank = jnp.dot(ltri, indf.astype(jnp.bfloat16),
                   preferred_element_type=jnp.float32)
    oh0f = oh0.astype(jnp.float32)
    oh1f = oh1.astype(jnp.float32)
    r0 = jnp.sum(rank * oh0f, axis=-1, keepdims=True)
    r1 = jnp.sum(rank * oh1f, axis=-1, keepdims=True)
    v0 = r0 < CAP
    v1 = r1 < CAP
    base0 = i0 * XSTRIDE
    base1 = i1 * XSTRIDE
    f0 = jnp.where(v0, base0 + r0.astype(jnp.int32), base0 + CAP)
    f1 = jnp.where(v1, base1 + r1.astype(jnp.int32), base1 + CAP)
    sw0 = jnp.where(v0, w0, 0.0)
    sw1 = jnp.where(v1, w1, 0.0)
    w0_ref[...] = jnp.broadcast_to(sw0, (S, 128))
    w1_ref[...] = jnp.broadcast_to(sw1, (S, 128))
    pk = jnp.where(lane == 0, f0.astype(jnp.float32),
                   jnp.where(lane == 1, f1.astype(jnp.float32), 0.0))
    rt_ref[...] = jnp.transpose(pk)[:8, :]
    counts = jnp.sum(indf, axis=0, keepdims=True)
    psum = jnp.sum(probs, axis=0, keepdims=True)
    nt = float(S)
    aux = E * jnp.sum(psum / nt * (counts / nt), axis=-1, keepdims=True)
    util = 100.0 * jnp.sum((counts > 0).astype(jnp.float32), axis=-1,
                           keepdims=True) / E
    row = lax.broadcasted_iota(jnp.int32, (8, 128), 0)
    st_ref[...] = jnp.where(row == 0, jnp.broadcast_to(aux, (8, 128)),
                            jnp.broadcast_to(util, (8, 128)))


def _k4(logits):
    return pl.pallas_call(
        _k4_body,
        out_shape=[
            jax.ShapeDtypeStruct((S, 128), jnp.float32),
            jax.ShapeDtypeStruct((S, 128), jnp.float32),
            jax.ShapeDtypeStruct((8, S), jnp.float32),
            jax.ShapeDtypeStruct((8, 128), jnp.float32),
        ],
    )(logits)


def _k6_body(x_ref, w1_ref, w2_ref, w3_ref, ws_ref, o_ref):
    x = x_ref[...]
    a = jnp.dot(x, w1_ref[0], preferred_element_type=jnp.float32)
    b = jnp.dot(x, w3_ref[0], preferred_element_type=jnp.float32)
    hsw = a * jax.nn.sigmoid(a) * b
    y = jnp.dot(hsw, w2_ref[0], preferred_element_type=jnp.float32)
    o_ref[...] = y * ws_ref[:, :1]


def _k6(xe, e_w1, e_w2, e_w3, w_slot):
    grid = (E,)
    return pl.pallas_call(
        _k6_body,
        grid=grid,
        in_specs=[
            pl.BlockSpec((XSTRIDE, D), lambda e: (e, 0)),
            pl.BlockSpec((1, D, HID), lambda e: (e, 0, 0)),
            pl.BlockSpec((1, HID, D), lambda e: (e, 0, 0)),
            pl.BlockSpec((1, D, HID), lambda e: (e, 0, 0)),
            pl.BlockSpec((XSTRIDE, 128), lambda e: (e, 0)),
        ],
        out_specs=pl.BlockSpec((XSTRIDE, D), lambda e: (e, 0)),
        out_shape=jax.ShapeDtypeStruct((NROWS, D), jnp.float32),
    )(xe, e_w1, e_w2, e_w3, w_slot)


def _k8_body(xn_ref, h_ref, m0_ref, m1_ref, w1_ref, w2_ref, w3_ref, o_ref):
    xn = xn_ref[...]
    a = jnp.dot(xn, w1_ref[...], preferred_element_type=jnp.float32)
    b = jnp.dot(xn, w3_ref[...], preferred_element_type=jnp.float32)
    hsw = a * jax.nn.sigmoid(a) * b
    y = jnp.dot(hsw, w2_ref[...], preferred_element_type=jnp.float32)
    o_ref[...] = h_ref[...] + m0_ref[...] + m1_ref[...] + y


def _k8(xn, h_out, m0, m1, s_w1, s_w2, s_w3):
    grid = (S // SB,)
    return pl.pallas_call(
        _k8_body,
        grid=grid,
        in_specs=[
            pl.BlockSpec((SB, D), lambda i: (i, 0)),
            pl.BlockSpec((SB, D), lambda i: (i, 0)),
            pl.BlockSpec((SB, D), lambda i: (i, 0)),
            pl.BlockSpec((SB, D), lambda i: (i, 0)),
            pl.BlockSpec((D, HID), lambda i: (0, 0)),
            pl.BlockSpec((HID, D), lambda i: (0, 0)),
            pl.BlockSpec((D, HID), lambda i: (0, 0)),
        ],
        out_specs=pl.BlockSpec((SB, D), lambda i: (i, 0)),
        out_shape=jax.ShapeDtypeStruct((S, D), jnp.float32),
    )(xn, h_out, m0, m1, s_w1, s_w2, s_w3)


from jax.experimental.pallas import tpu_sc as plsc

NSC, NSUB = 2, 16
NW = NSC * NSUB
TPW = S // NW
NG = TPW // 16


def _sc_mesh():
    return plsc.VectorSubcoreMesh(core_axis_name="c", subcore_axis_name="s",
                                  num_cores=NSC, num_subcores=NSUB)


def _load_idx(rt_hbm, row, base, r_v, i_v):
    pltpu.sync_copy(rt_hbm.at[pl.ds(row * S + base, TPW)], r_v)
    for g in range(NG):
        i_v[pl.ds(g * 16, 16)] = r_v[pl.ds(g * 16, 16)].astype(jnp.int32)


def _sc_dispatch_body(xn_hbm, w0_hbm, w1_hbm, rt_hbm, xe_hbm, ws_hbm,
                      xn_v, w0_v, w1_v, r_v, i0_v, i1_v, sem):
    wid = lax.axis_index("s") * NSC + lax.axis_index("c")
    base = wid * TPW
    pltpu.sync_copy(xn_hbm.at[pl.ds(base, TPW)], xn_v)
    pltpu.sync_copy(w0_hbm.at[pl.ds(base, TPW)], w0_v)
    pltpu.sync_copy(w1_hbm.at[pl.ds(base, TPW)], w1_v)
    _load_idx(rt_hbm, 0, base, r_v, i0_v)
    _load_idx(rt_hbm, 1, base, r_v, i1_v)
    pltpu.async_copy(xn_v, xe_hbm.at[i0_v], sem).wait()
    pltpu.async_copy(xn_v, xe_hbm.at[i1_v], sem).wait()
    pltpu.async_copy(w0_v, ws_hbm.at[i0_v], sem).wait()
    pltpu.async_copy(w1_v, ws_hbm.at[i1_v], sem).wait()


def _dispatch(xn, w0b, w1b, rt_flat):
    return pl.kernel(
        _sc_dispatch_body,
        out_type=[
            jax.ShapeDtypeStruct((NROWS, D), jnp.float32),
            jax.ShapeDtypeStruct((NROWS, 128), jnp.float32),
        ],
        mesh=_sc_mesh(),
        scratch_types=[
            pltpu.VMEM((TPW, D), jnp.float32),
            pltpu.VMEM((TPW, 128), jnp.float32),
            pltpu.VMEM((TPW, 128), jnp.float32),
            pltpu.VMEM((TPW,), jnp.float32),
            pltpu.VMEM((TPW,), jnp.int32),
            pltpu.VMEM((TPW,), jnp.int32),
            pltpu.SemaphoreType.DMA,
        ],
    )(xn, w0b, w1b, rt_flat)


def _sc_combine_body(ye_hbm, rt_hbm, m0_hbm, m1_hbm,
                     r_v, i0_v, i1_v, g_v, sem):
    wid = lax.axis_index("s") * NSC + lax.axis_index("c")
    base = wid * TPW
    _load_idx(rt_hbm, 0, base, r_v, i0_v)
    _load_idx(rt_hbm, 1, base, r_v, i1_v)
    half = TPW // 2
    for h in range(2):
        pltpu.async_copy(ye_hbm.at[i0_v.at[pl.ds(h * half, half)]], g_v,
                         sem).wait()
        pltpu.sync_copy(g_v, m0_hbm.at[pl.ds(base + h * half, half)])
        pltpu.async_copy(ye_hbm.at[i1_v.at[pl.ds(h * half, half)]], g_v,
                         sem).wait()
        pltpu.sync_copy(g_v, m1_hbm.at[pl.ds(base + h * half, half)])


def _combine(ye, rt_flat):
    half = TPW // 2
    return pl.kernel(
        _sc_combine_body,
        out_type=[
            jax.ShapeDtypeStruct((S, D), jnp.float32),
            jax.ShapeDtypeStruct((S, D), jnp.float32),
        ],
        mesh=_sc_mesh(),
        scratch_types=[
            pltpu.VMEM((TPW,), jnp.float32),
            pltpu.VMEM((TPW,), jnp.int32),
            pltpu.VMEM((TPW,), jnp.int32),
            pltpu.VMEM((half, D), jnp.float32),
            pltpu.SemaphoreType.DMA,
        ],
    )(ye, rt_flat)


def kernel(x_in, cos, sin, mask, attn_norm_w, wq, wk, wv, wo, moe_norm_w,
           router_w, e_w1, e_w2, e_w3, s_w1, s_w2, s_w3):
    x = x_in.reshape(S, D)
    c32 = cos[0, :, 0, :32]
    s32 = sin[0, :, 0, :32]
    c64 = jnp.concatenate([c32, c32], axis=1)
    s64 = jnp.concatenate([-s32, s32], axis=1)
    cq = jnp.tile(c64, (1, NH))
    sq = jnp.tile(s64, (1, NH))
    ck = jnp.tile(c64, (1, NKV))
    sk = jnp.tile(s64, (1, NKV))
    nw_a = attn_norm_w.reshape(1, D)
    nw_m = moe_norm_w.reshape(1, D)
    rw_pad = jnp.pad(router_w, ((0, 0), (0, 128 - E)))

    ar_q = jnp.arange(NH * HD)
    rq = (
        (ar_q[:, None] ^ 32) == ar_q[None, :]).astype(jnp.bfloat16)
    ar_k = jnp.arange(NKV * HD)
    rk = (
        (ar_k[:, None] ^ 32) == ar_k[None, :]).astype(jnp.bfloat16)

    q_hm, k_hm, v_hm = _k1(x, wq, wk, wv, nw_a, cq, sq, ck, sk, rq, rk)

    new_k = k_hm.reshape(1, S, NKV, HD)
    new_v = v_hm.reshape(1, S, NKV, HD)

    ao = _k2(q_hm, k_hm, v_hm)

    h_out, xn, logits = _k3(ao, x, wo, nw_m, rw_pad)

    w0b, w1b, rt, stats = _k4(logits)
    rt_flat = rt.reshape(8 * S)

    xe, w_slot = _dispatch(xn, w0b, w1b, rt_flat)

    ye = _k6(xe, e_w1, e_w2, e_w3, w_slot)

    m0, m1 = _combine(ye, rt_flat)

    out = _k8(xn, h_out, m0, m1, s_w1, s_w2, s_w3)

    return (out.reshape(1, S, D), stats[0, 0], stats[1, 0], new_k, new_v)
